# baseline (speedup 1.0000x reference)
import jax
import jax.numpy as jnp
from jax.experimental import pallas as pl
from jax.experimental.pallas import tpu as pltpu

_B, _ROI, _T = 32, 180, 64
_L = 3
_H2 = _T // 2
_N = _B * _ROI
_OPAD = 128


def _bn(v, gamma, beta, eps=1e-5):
    mu = jnp.mean(v, axis=0, keepdims=True)
    var = jnp.mean((v - mu) * (v - mu), axis=0, keepdims=True)
    return gamma * (v - mu) * jax.lax.rsqrt(var + eps) + beta


def _elu(v):
    return jnp.where(v > 0, v, jnp.exp(v) - 1.0)


def _matmul_t(a, w):
    return jax.lax.dot_general(a, w, (((1,), (1,)), ((), ())),
                               preferred_element_type=jnp.float32)


def _fused(x_ref, A_ref, W1_ref, b1_ref, g1_ref, be1_ref, W2_ref, b2_ref,
           g2_ref, be2_ref, eps_ref, gbn_ref, bbn_ref, Wm1_ref, bm1_ref,
           gm_ref, bm_ref, Wm2_ref, bm2_ref, out_ref, xf_ref, agg_ref):
    xf_ref[...] = x_ref[...]

    def agg_body(i, _):
        m = (A_ref[i] != 0).astype(jnp.float32)
        agg_ref[i] = jnp.dot(m, xf_ref[i], preferred_element_type=jnp.float32)
        return 0

    for l in range(_L):
        jax.lax.fori_loop(0, _B, agg_body, 0, unroll=True)
        xf = xf_ref[...].reshape(_N, _T)
        agg = agg_ref[...].reshape(_N, _T)
        v = agg + eps_ref[l] * xf
        h = _elu(_bn(_matmul_t(v, W1_ref[l]) + b1_ref[l], g1_ref[l], be1_ref[l]))
        h = _elu(_bn(_matmul_t(h, W2_ref[l]) + b2_ref[l], g2_ref[l], be2_ref[l]))
        xf = _elu(_bn(h, gbn_ref[l], bbn_ref[l]))
        xf_ref[...] = xf.reshape(_B, _ROI, _T)

    xm = jnp.mean(xf_ref[...], axis=1)
    m1 = _bn(_matmul_t(xm, Wm1_ref[...]) + bm1_ref[...], gm_ref[...], bm_ref[...])
    m1 = jnp.maximum(m1, 0.0)
    out_ref[...] = _matmul_t(m1, Wm2_ref[...]) + bm2_ref[...]


def kernel(x, A, W1, b1, g1, be1, W2, b2, g2, be2, eps_p, gbn, bbn,
           Wm1, bm1, gm, bm, Wm2, bm2):
    Wm2p = jnp.zeros((_OPAD, _H2), jnp.float32).at[:2].set(Wm2)
    bm2p = jnp.zeros((1, _OPAD), jnp.float32).at[0, :2].set(bm2)

    args = (
        x, A, W1,
        b1.reshape(_L, 1, _T), g1.reshape(_L, 1, _T), be1.reshape(_L, 1, _T),
        W2,
        b2.reshape(_L, 1, _T), g2.reshape(_L, 1, _T), be2.reshape(_L, 1, _T),
        eps_p.reshape(_L, 1, 1),
        gbn.reshape(_L, 1, _T), bbn.reshape(_L, 1, _T),
        Wm1,
        bm1.reshape(1, _H2), gm.reshape(1, _H2), bm.reshape(1, _H2),
        Wm2p, bm2p,
    )

    out = pl.pallas_call(
        _fused,
        out_shape=jax.ShapeDtypeStruct((_B, _OPAD), jnp.float32),
        scratch_shapes=[
            pltpu.VMEM((_B, _ROI, _T), jnp.float32),
            pltpu.VMEM((_B, _ROI, _T), jnp.float32),
        ],
    )(*args)
    return out[:, :2]
